# baseline (speedup 1.0000x reference)
import functools

import jax
import jax.numpy as jnp
from jax import lax
from jax.experimental import pallas as pl
from jax.experimental.pallas import tpu as pltpu
from jax.experimental.pallas import tpu_sc as plsc

VOCAB = 100000
EMBED_DIM = 16
BATCH = 1024
CTX = 20

_INFO = plsc.get_sparse_core_info()
_NC, _NS = _INFO.num_cores, _INFO.num_subcores
_NW = _NC * _NS
_BPW = BATCH // _NW


def _make_gather_mean():
    mesh = plsc.VectorSubcoreMesh(core_axis_name="c", subcore_axis_name="s")

    @functools.partial(
        pl.kernel,
        mesh=mesh,
        out_type=jax.ShapeDtypeStruct((BATCH, EMBED_DIM), jnp.float32),
        scratch_types=[
            pltpu.VMEM((_BPW * CTX,), jnp.int32),
            pltpu.VMEM((_BPW * CTX, EMBED_DIM), jnp.float32),
            pltpu.VMEM((_BPW, EMBED_DIM), jnp.float32),
            pltpu.SemaphoreType.DMA,
        ],
        compiler_params=pltpu.CompilerParams(use_tc_tiling_on_sc=False),
    )
    def gather_mean(ctx_hbm, table_hbm, out_hbm, idx_v, rows_v, out_v, sem):
        wid = lax.axis_index("s") * _NC + lax.axis_index("c")
        base = wid * _BPW
        pltpu.sync_copy(ctx_hbm.at[pl.ds(base * CTX, _BPW * CTX)], idx_v)
        pltpu.async_copy(table_hbm.at[idx_v], rows_v, sem).wait()

        def body(b, _):
            acc = rows_v[b * CTX]
            for j in range(1, CTX):
                acc = acc + rows_v[b * CTX + j]
            out_v[b] = acc * (1.0 / CTX)
            return 0

        lax.fori_loop(0, _BPW, body, 0)
        pltpu.sync_copy(out_v, out_hbm.at[pl.ds(base, _BPW)])

    return gather_mean


_gather_mean = _make_gather_mean()

_BT = 32
_NBT = BATCH // _BT
_NQ = 4
_RQ = _BT // _NQ


_TCH = 4000


def _mm_body(avg_ref, w_hbm, b_ref, out_hbm, buf, wt, wch, wsem, sems):
    i = pl.program_id(0)
    slot = i % 2

    @pl.when(i == 0)
    def _():
        nch = VOCAB // _TCH
        pltpu.make_async_copy(
            w_hbm.at[pl.ds(0, _TCH), :], wch.at[0], wsem.at[0]
        ).start()
        for c in range(nch):
            cs = c % 2
            if c + 1 < nch:
                pltpu.make_async_copy(
                    w_hbm.at[pl.ds((c + 1) * _TCH, _TCH), :],
                    wch.at[1 - cs],
                    wsem.at[1 - cs],
                ).start()
            pltpu.make_async_copy(
                w_hbm.at[pl.ds(c * _TCH, _TCH), :], wch.at[cs], wsem.at[cs]
            ).wait()
            wt[:, pl.ds(c * _TCH, _TCH)] = jnp.transpose(wch[cs], (1, 0))

    prod = (
        lax.dot_general(
            avg_ref[...],
            wt[...],
            (((1,), (0,)), ((), ())),
            preferred_element_type=jnp.float32,
            precision=lax.Precision.DEFAULT,
        )
        + b_ref[...]
    )

    def copies(s, step):
        return [
            pltpu.make_async_copy(
                buf.at[s, pl.ds(q * _RQ, _RQ), :],
                out_hbm.at[pl.ds(step * _BT + q * _RQ, _RQ), :],
                sems.at[s, q],
            )
            for q in range(_NQ)
        ]

    @pl.when(i >= 2)
    def _():
        for c in copies(slot, i - 2):
            c.wait()

    buf[slot] = prod
    for c in copies(slot, i):
        c.start()

    @pl.when(i == _NBT - 1)
    def _():
        for c in copies(1 - slot, i - 1):
            c.wait()
        for c in copies(slot, i):
            c.wait()


def _projection(avg, W, b2d):
    return pl.pallas_call(
        _mm_body,
        grid=(_NBT,),
        in_specs=[
            pl.BlockSpec((_BT, EMBED_DIM), lambda i: (i, 0)),
            pl.BlockSpec(memory_space=pltpu.MemorySpace.HBM),
            pl.BlockSpec((1, VOCAB), lambda i: (0, 0)),
        ],
        out_specs=pl.BlockSpec(memory_space=pltpu.MemorySpace.HBM),
        out_shape=jax.ShapeDtypeStruct((BATCH, VOCAB), jnp.float32),
        scratch_shapes=[
            pltpu.VMEM((2, _BT, VOCAB), jnp.float32),
            pltpu.VMEM((EMBED_DIM, VOCAB), jnp.float32),
            pltpu.VMEM((2, _TCH, EMBED_DIM), jnp.float32),
            pltpu.SemaphoreType.DMA((2,)),
            pltpu.SemaphoreType.DMA((2, _NQ)),
        ],
        compiler_params=pltpu.CompilerParams(
            dimension_semantics=("arbitrary",),
            vmem_limit_bytes=100 * 1024 * 1024,
        ),
    )(avg, W, b2d)


@jax.jit
def kernel(context_words, in_emb, W, b):
    ctx_flat = context_words.reshape(-1).astype(jnp.int32)
    avg = _gather_mean(ctx_flat, in_emb)
    return _projection(avg, W, b.reshape(1, VOCAB))


# speedup vs baseline: 2.4138x; 2.4138x over previous
import functools

import jax
import jax.numpy as jnp
from jax import lax
from jax.experimental import pallas as pl
from jax.experimental.pallas import tpu as pltpu
from jax.experimental.pallas import tpu_sc as plsc

VOCAB = 100000
EMBED_DIM = 16
BATCH = 1024
CTX = 20

_INFO = plsc.get_sparse_core_info()
_NC, _NS = _INFO.num_cores, _INFO.num_subcores
_NW = _NC * _NS
_BPW = BATCH // _NW


def _make_gather_mean():
    mesh = plsc.VectorSubcoreMesh(core_axis_name="c", subcore_axis_name="s")

    @functools.partial(
        pl.kernel,
        mesh=mesh,
        out_type=jax.ShapeDtypeStruct((BATCH, EMBED_DIM), jnp.float32),
        scratch_types=[
            pltpu.VMEM((_BPW * CTX,), jnp.int32),
            pltpu.VMEM((_BPW * CTX, EMBED_DIM), jnp.float32),
            pltpu.VMEM((_BPW, EMBED_DIM), jnp.float32),
            pltpu.SemaphoreType.DMA,
        ],
        compiler_params=pltpu.CompilerParams(use_tc_tiling_on_sc=False),
    )
    def gather_mean(ctx_hbm, table_hbm, out_hbm, idx_v, rows_v, out_v, sem):
        wid = lax.axis_index("s") * _NC + lax.axis_index("c")
        base = wid * _BPW
        pltpu.sync_copy(ctx_hbm.at[pl.ds(base * CTX, _BPW * CTX)], idx_v)
        pltpu.async_copy(table_hbm.at[idx_v], rows_v, sem).wait()

        def body(b, _):
            acc = rows_v[b * CTX]
            for j in range(1, CTX):
                acc = acc + rows_v[b * CTX + j]
            out_v[b] = acc * (1.0 / CTX)
            return 0

        lax.fori_loop(0, _BPW, body, 0)
        pltpu.sync_copy(out_v, out_hbm.at[pl.ds(base, _BPW)])

    return gather_mean


_gather_mean = _make_gather_mean()

_VT = 4096
_NVT = (VOCAB + _VT - 1) // _VT


def _mm_body(wt_ref, avg_ref, b_ref, outt_ref):
    outt_ref[...] = (
        lax.dot_general(
            wt_ref[...],
            avg_ref[...],
            (((0,), (1,)), ((), ())),
            preferred_element_type=jnp.float32,
            precision=lax.Precision.DEFAULT,
        )
        + b_ref[...]
    )


def _projection(avg, Wt, bcol):
    return pl.pallas_call(
        _mm_body,
        grid=(_NVT,),
        in_specs=[
            pl.BlockSpec((EMBED_DIM, _VT), lambda i: (0, i)),
            pl.BlockSpec((BATCH, EMBED_DIM), lambda i: (0, 0)),
            pl.BlockSpec((_VT, 1), lambda i: (i, 0)),
        ],
        out_specs=pl.BlockSpec((_VT, BATCH), lambda i: (i, 0)),
        out_shape=jax.ShapeDtypeStruct((VOCAB, BATCH), jnp.float32),
        compiler_params=pltpu.CompilerParams(
            dimension_semantics=("arbitrary",),
        ),
    )(Wt, avg, bcol)


@jax.jit
def kernel(context_words, in_emb, W, b):
    ctx_flat = context_words.reshape(-1).astype(jnp.int32)
    avg = _gather_mean(ctx_flat, in_emb)
    logits_t = _projection(avg, W.T, b.reshape(VOCAB, 1))
    return logits_t.T


# speedup vs baseline: 3.0741x; 1.2736x over previous
import functools

import jax
import jax.numpy as jnp
from jax import lax
from jax.experimental import pallas as pl
from jax.experimental.pallas import tpu as pltpu
from jax.experimental.pallas import tpu_sc as plsc

VOCAB = 100000
EMBED_DIM = 16
BATCH = 1024
CTX = 20

_INFO = plsc.get_sparse_core_info()
_NC, _NS = _INFO.num_cores, _INFO.num_subcores
_NW = _NC * _NS
_BPW = BATCH // _NW


def _make_gather_mean():
    mesh = plsc.VectorSubcoreMesh(core_axis_name="c", subcore_axis_name="s")

    @functools.partial(
        pl.kernel,
        mesh=mesh,
        out_type=jax.ShapeDtypeStruct((BATCH, EMBED_DIM), jnp.float32),
        scratch_types=[
            pltpu.VMEM((_BPW * CTX,), jnp.int32),
            pltpu.VMEM((_BPW * CTX, EMBED_DIM), jnp.float32),
            pltpu.VMEM((_BPW, EMBED_DIM), jnp.float32),
            pltpu.SemaphoreType.DMA,
        ],
        compiler_params=pltpu.CompilerParams(use_tc_tiling_on_sc=False),
    )
    def gather_mean(ctx_hbm, table_hbm, out_hbm, idx_v, rows_v, out_v, sem):
        wid = lax.axis_index("s") * _NC + lax.axis_index("c")
        base = wid * _BPW
        pltpu.sync_copy(ctx_hbm.at[pl.ds(base * CTX, _BPW * CTX)], idx_v)
        pltpu.async_copy(table_hbm.at[idx_v], rows_v, sem).wait()

        def body(b, _):
            acc = rows_v[b * CTX]
            for j in range(1, CTX):
                acc = acc + rows_v[b * CTX + j]
            out_v[b] = acc * (1.0 / CTX)
            return 0

        lax.fori_loop(0, _BPW, body, 0)
        pltpu.sync_copy(out_v, out_hbm.at[pl.ds(base, _BPW)])

    return gather_mean


_gather_mean = _make_gather_mean()

_VT = 4096
_NVT = (VOCAB + _VT - 1) // _VT


def _mm_body(wt_ref, avg_ref, b_ref, outt_ref):
    wtb = jnp.concatenate([wt_ref[...], b_ref[...]], axis=0)
    avg1 = jnp.concatenate(
        [avg_ref[...], jnp.ones((BATCH, 1), jnp.float32)], axis=1
    )
    outt_ref[...] = lax.dot_general(
        wtb,
        avg1,
        (((0,), (1,)), ((), ())),
        preferred_element_type=jnp.float32,
        precision=lax.Precision.DEFAULT,
    )


def _projection(avg, Wt, bcol):
    return pl.pallas_call(
        _mm_body,
        grid=(_NVT,),
        in_specs=[
            pl.BlockSpec((EMBED_DIM, _VT), lambda i: (0, i)),
            pl.BlockSpec((BATCH, EMBED_DIM), lambda i: (0, 0)),
            pl.BlockSpec((1, _VT), lambda i: (0, i)),
        ],
        out_specs=pl.BlockSpec((_VT, BATCH), lambda i: (i, 0)),
        out_shape=jax.ShapeDtypeStruct((VOCAB, BATCH), jnp.float32),
        compiler_params=pltpu.CompilerParams(
            dimension_semantics=("arbitrary",),
        ),
    )(Wt, avg, bcol)


@jax.jit
def kernel(context_words, in_emb, W, b):
    ctx_flat = context_words.reshape(-1).astype(jnp.int32)
    avg = _gather_mean(ctx_flat, in_emb)
    logits_t = _projection(avg, W.T, b.reshape(1, VOCAB))
    return logits_t.T


# speedup vs baseline: 3.0944x; 1.0066x over previous
import functools

import jax
import jax.numpy as jnp
from jax import lax
from jax.experimental import pallas as pl
from jax.experimental.pallas import tpu as pltpu
from jax.experimental.pallas import tpu_sc as plsc

VOCAB = 100000
EMBED_DIM = 16
BATCH = 1024
CTX = 20

_INFO = plsc.get_sparse_core_info()
_NC, _NS = _INFO.num_cores, _INFO.num_subcores
_NW = _NC * _NS
_BPW = BATCH // _NW


def _make_gather_mean():
    mesh = plsc.VectorSubcoreMesh(core_axis_name="c", subcore_axis_name="s")

    @functools.partial(
        pl.kernel,
        mesh=mesh,
        out_type=jax.ShapeDtypeStruct((BATCH, EMBED_DIM), jnp.float32),
        scratch_types=[
            pltpu.VMEM((_BPW * CTX,), jnp.int32),
            pltpu.VMEM((_BPW * CTX, EMBED_DIM), jnp.float32),
            pltpu.VMEM((_BPW, EMBED_DIM), jnp.float32),
            pltpu.SemaphoreType.DMA,
        ],
        compiler_params=pltpu.CompilerParams(use_tc_tiling_on_sc=False),
    )
    def gather_mean(ctx_hbm, table_hbm, out_hbm, idx_v, rows_v, out_v, sem):
        wid = lax.axis_index("s") * _NC + lax.axis_index("c")
        base = wid * _BPW
        pltpu.sync_copy(ctx_hbm.at[pl.ds(base * CTX, _BPW * CTX)], idx_v)
        pltpu.async_copy(table_hbm.at[idx_v], rows_v, sem).wait()

        def body(b, _):
            acc = rows_v[b * CTX]
            for j in range(1, CTX):
                acc = acc + rows_v[b * CTX + j]
            out_v[b] = acc * (1.0 / CTX)
            return 0

        lax.fori_loop(0, _BPW, body, 0)
        pltpu.sync_copy(out_v, out_hbm.at[pl.ds(base, _BPW)])

    return gather_mean


_gather_mean = _make_gather_mean()

_VT = 2048
_NVT = (VOCAB + _VT - 1) // _VT


def _mm_body(wt_ref, avg_ref, b_ref, outt_ref):
    wtb = jnp.concatenate([wt_ref[...], b_ref[...]], axis=0)
    avg1 = jnp.concatenate(
        [avg_ref[...], jnp.ones((BATCH, 1), jnp.float32)], axis=1
    )
    outt_ref[...] = lax.dot_general(
        wtb,
        avg1,
        (((0,), (1,)), ((), ())),
        preferred_element_type=jnp.float32,
        precision=lax.Precision.DEFAULT,
    )


def _projection(avg, Wt, bcol):
    return pl.pallas_call(
        _mm_body,
        grid=(_NVT,),
        in_specs=[
            pl.BlockSpec((EMBED_DIM, _VT), lambda i: (0, i)),
            pl.BlockSpec((BATCH, EMBED_DIM), lambda i: (0, 0)),
            pl.BlockSpec((1, _VT), lambda i: (0, i)),
        ],
        out_specs=pl.BlockSpec((_VT, BATCH), lambda i: (i, 0)),
        out_shape=jax.ShapeDtypeStruct((VOCAB, BATCH), jnp.float32),
        compiler_params=pltpu.CompilerParams(
            dimension_semantics=("arbitrary",),
        ),
    )(Wt, avg, bcol)


@jax.jit
def kernel(context_words, in_emb, W, b):
    ctx_flat = context_words.reshape(-1).astype(jnp.int32)
    avg = _gather_mean(ctx_flat, in_emb)
    logits_t = _projection(avg, W.T, b.reshape(1, VOCAB))
    return logits_t.T
